# baseline (speedup 1.0000x reference)
import jax
import jax.numpy as jnp
from jax.experimental import pallas as pl
from jax.experimental.pallas import tpu as pltpu


def _bf(x):
    return x.astype(jnp.bfloat16)


def _fw_kernel(f_ref, w_ref, o_ref):
    o_ref[...] = jnp.dot(_bf(f_ref[...]), _bf(w_ref[...]),
                         preferred_element_type=jnp.float32)


def _agg_kernel(d_ref, c_ref, fw_ref, w2_ref, bcat_ref, y_ref, acc_ref):
    j = pl.program_id(1)

    @pl.when(j == 0)
    def _init():
        acc_ref[...] = jnp.zeros_like(acc_ref)

    d = d_ref[...]
    cd = c_ref[...] * d
    fw = fw_ref[...]
    h = fw.shape[1] // 2
    acc_ref[:, :h] += jnp.dot(_bf(d), _bf(fw[:, :h]),
                              preferred_element_type=jnp.float32)
    acc_ref[:, h:] += jnp.dot(_bf(cd), _bf(fw[:, h:]),
                              preferred_element_type=jnp.float32)

    @pl.when(j == pl.num_programs(1) - 1)
    def _epilogue():
        x = jnp.maximum(acc_ref[...] + bcat_ref[...], 0.0)
        y_ref[...] = jnp.dot(_bf(x), _bf(w2_ref[...]),
                             preferred_element_type=jnp.float32)


def _out_kernel(d_ref, y_ref, b2_ref, o_ref):
    j = pl.program_id(1)

    @pl.when(j == 0)
    def _init():
        o_ref[...] = jnp.broadcast_to(b2_ref[...], o_ref.shape)

    o_ref[...] += jnp.dot(_bf(d_ref[...]), _bf(y_ref[...]),
                          preferred_element_type=jnp.float32)


def kernel(features, dist, adj_relative_cos, W1, b1, Wa, ba, W2, b2):
    n, in_dim = features.shape
    hid = W1.shape[1]
    out_dim = W2.shape[1]

    wcat = jnp.concatenate([W1, Wa], axis=1)
    bcat = jnp.concatenate([b1, ba]).reshape(1, -1)
    b2r = b2.reshape(1, -1)
    two_h = hid + in_dim

    bi1 = 512
    fw = pl.pallas_call(
        _fw_kernel,
        grid=(n // bi1,),
        in_specs=[
            pl.BlockSpec((bi1, in_dim), lambda i: (i, 0)),
            pl.BlockSpec((in_dim, two_h), lambda i: (0, 0)),
        ],
        out_specs=pl.BlockSpec((bi1, two_h), lambda i: (i, 0)),
        out_shape=jax.ShapeDtypeStruct((n, two_h), jnp.float32),
        compiler_params=pltpu.CompilerParams(
            dimension_semantics=("parallel",)),
    )(features, wcat)

    bi2, bj2 = 256, 512
    y = pl.pallas_call(
        _agg_kernel,
        grid=(n // bi2, n // bj2),
        in_specs=[
            pl.BlockSpec((bi2, bj2), lambda i, j: (i, j)),
            pl.BlockSpec((bi2, bj2), lambda i, j: (i, j)),
            pl.BlockSpec((bj2, two_h), lambda i, j: (j, 0)),
            pl.BlockSpec((two_h, out_dim), lambda i, j: (0, 0)),
            pl.BlockSpec((1, two_h), lambda i, j: (0, 0)),
        ],
        out_specs=pl.BlockSpec((bi2, out_dim), lambda i, j: (i, 0)),
        out_shape=jax.ShapeDtypeStruct((n, out_dim), jnp.float32),
        scratch_shapes=[pltpu.VMEM((bi2, two_h), jnp.float32)],
        compiler_params=pltpu.CompilerParams(
            dimension_semantics=("parallel", "arbitrary")),
    )(dist, adj_relative_cos, fw, W2, bcat)

    bi3, bj3 = 256, 512
    out = pl.pallas_call(
        _out_kernel,
        grid=(n // bi3, n // bj3),
        in_specs=[
            pl.BlockSpec((bi3, bj3), lambda i, j: (i, j)),
            pl.BlockSpec((bj3, out_dim), lambda i, j: (j, 0)),
            pl.BlockSpec((1, out_dim), lambda i, j: (0, 0)),
        ],
        out_specs=pl.BlockSpec((bi3, out_dim), lambda i, j: (i, 0)),
        out_shape=jax.ShapeDtypeStruct((n, out_dim), jnp.float32),
        compiler_params=pltpu.CompilerParams(
            dimension_semantics=("parallel", "arbitrary")),
    )(dist, y, b2r)

    return out


# speedup vs baseline: 2.8302x; 2.8302x over previous
import jax
import jax.numpy as jnp
from jax.experimental import pallas as pl
from jax.experimental.pallas import tpu as pltpu


def _bf(x):
    return x.astype(jnp.bfloat16)


def _fw_kernel(f_ref, w_ref, o_ref):
    o_ref[...] = jnp.dot(_bf(f_ref[...]), _bf(w_ref[...]),
                         preferred_element_type=jnp.float32)


def _agg_kernel(d_ref, c_ref, fw_ref, w2_ref, bcat_ref, y_ref):
    d = d_ref[...]
    cd = c_ref[...] * d
    fw = fw_ref[...]
    h = fw.shape[1] // 2
    x1 = jnp.dot(_bf(d), _bf(fw[:, :h]), preferred_element_type=jnp.float32)
    x2 = jnp.dot(_bf(cd), _bf(fw[:, h:]), preferred_element_type=jnp.float32)
    x = jnp.maximum(jnp.concatenate([x1, x2], axis=1) + bcat_ref[...], 0.0)
    y_ref[...] = jnp.dot(_bf(x), _bf(w2_ref[...]),
                         preferred_element_type=jnp.float32)


def _out_kernel(d_ref, y_ref, b2_ref, o_ref):
    o_ref[...] = jnp.dot(_bf(d_ref[...]), _bf(y_ref[...]),
                         preferred_element_type=jnp.float32) + b2_ref[...]


def kernel(features, dist, adj_relative_cos, W1, b1, Wa, ba, W2, b2):
    n, in_dim = features.shape
    hid = W1.shape[1]
    out_dim = W2.shape[1]

    wcat = jnp.concatenate([W1, Wa], axis=1)
    bcat = jnp.concatenate([b1, ba]).reshape(1, -1)
    b2r = b2.reshape(1, -1)
    two_h = hid + in_dim

    bi1 = 512
    fw = pl.pallas_call(
        _fw_kernel,
        grid=(n // bi1,),
        in_specs=[
            pl.BlockSpec((bi1, in_dim), lambda i: (i, 0)),
            pl.BlockSpec((in_dim, two_h), lambda i: (0, 0)),
        ],
        out_specs=pl.BlockSpec((bi1, two_h), lambda i: (i, 0)),
        out_shape=jax.ShapeDtypeStruct((n, two_h), jnp.float32),
        compiler_params=pltpu.CompilerParams(
            dimension_semantics=("arbitrary",)),
    )(features, wcat)

    bi2 = 256
    y = pl.pallas_call(
        _agg_kernel,
        grid=(n // bi2,),
        in_specs=[
            pl.BlockSpec((bi2, n), lambda i: (i, 0)),
            pl.BlockSpec((bi2, n), lambda i: (i, 0)),
            pl.BlockSpec((n, two_h), lambda i: (0, 0)),
            pl.BlockSpec((two_h, out_dim), lambda i: (0, 0)),
            pl.BlockSpec((1, two_h), lambda i: (0, 0)),
        ],
        out_specs=pl.BlockSpec((bi2, out_dim), lambda i: (i, 0)),
        out_shape=jax.ShapeDtypeStruct((n, out_dim), jnp.float32),
        compiler_params=pltpu.CompilerParams(
            dimension_semantics=("arbitrary",)),
    )(dist, adj_relative_cos, fw, W2, bcat)

    bi3 = 256
    out = pl.pallas_call(
        _out_kernel,
        grid=(n // bi3,),
        in_specs=[
            pl.BlockSpec((bi3, n), lambda i: (i, 0)),
            pl.BlockSpec((n, out_dim), lambda i: (0, 0)),
            pl.BlockSpec((1, out_dim), lambda i: (0, 0)),
        ],
        out_specs=pl.BlockSpec((bi3, out_dim), lambda i: (i, 0)),
        out_shape=jax.ShapeDtypeStruct((n, out_dim), jnp.float32),
        compiler_params=pltpu.CompilerParams(
            dimension_semantics=("arbitrary",)),
    )(dist, y, b2r)

    return out


# speedup vs baseline: 2.9513x; 1.0428x over previous
import jax
import jax.numpy as jnp
from jax.experimental import pallas as pl
from jax.experimental.pallas import tpu as pltpu


def _bf(x):
    return x.astype(jnp.bfloat16)


def _fused_kernel(f_ref, d_ref, c_ref, w_ref, w2_ref, bcat_ref, b2_ref,
                  o_ref, fw_ref, y_ref):
    p = pl.program_id(0)
    i = pl.program_id(1)
    bi = f_ref.shape[0]
    h = fw_ref.shape[1] // 2

    @pl.when(p == 0)
    def _phase_fw():
        fw_ref[pl.ds(i * bi, bi), :] = _bf(
            jnp.dot(_bf(f_ref[...]), _bf(w_ref[...]),
                    preferred_element_type=jnp.float32))

    @pl.when(p == 1)
    def _phase_agg():
        d = d_ref[...]
        cd = c_ref[...] * d
        x1 = jnp.dot(_bf(d), fw_ref[:, :h],
                     preferred_element_type=jnp.float32)
        x2 = jnp.dot(_bf(cd), fw_ref[:, h:],
                     preferred_element_type=jnp.float32)
        x = jnp.maximum(jnp.concatenate([x1, x2], axis=1) + bcat_ref[...],
                        0.0)
        y_ref[pl.ds(i * bi, bi), :] = _bf(
            jnp.dot(_bf(x), _bf(w2_ref[...]),
                    preferred_element_type=jnp.float32))

    @pl.when(p == 2)
    def _phase_out():
        o_ref[...] = jnp.dot(_bf(d_ref[...]), y_ref[...],
                             preferred_element_type=jnp.float32) + b2_ref[...]


def kernel(features, dist, adj_relative_cos, W1, b1, Wa, ba, W2, b2):
    n, in_dim = features.shape
    hid = W1.shape[1]
    out_dim = W2.shape[1]
    two_h = hid + in_dim

    wcat = jnp.concatenate([W1, Wa], axis=1)
    bcat = jnp.concatenate([b1, ba]).reshape(1, -1)
    b2r = b2.reshape(1, -1)

    bi = 256
    steps = n // bi

    out = pl.pallas_call(
        _fused_kernel,
        grid=(3, steps),
        in_specs=[
            pl.BlockSpec((bi, in_dim),
                         lambda p, i: (jnp.where(p == 0, i, 0), 0)),
            pl.BlockSpec((bi, n),
                         lambda p, i: (jnp.where(p == 0, 0, i), 0)),
            pl.BlockSpec((bi, n),
                         lambda p, i: (jnp.where(p == 1, i, 0), 0)),
            pl.BlockSpec((in_dim, two_h), lambda p, i: (0, 0)),
            pl.BlockSpec((two_h, out_dim), lambda p, i: (0, 0)),
            pl.BlockSpec((1, two_h), lambda p, i: (0, 0)),
            pl.BlockSpec((1, out_dim), lambda p, i: (0, 0)),
        ],
        out_specs=pl.BlockSpec((bi, out_dim),
                               lambda p, i: (jnp.where(p == 2, i, 0), 0)),
        out_shape=jax.ShapeDtypeStruct((n, out_dim), jnp.float32),
        scratch_shapes=[
            pltpu.VMEM((n, two_h), jnp.bfloat16),
            pltpu.VMEM((n, out_dim), jnp.bfloat16),
        ],
        compiler_params=pltpu.CompilerParams(
            dimension_semantics=("arbitrary", "arbitrary")),
    )(features, dist, adj_relative_cos, wcat, W2, bcat, b2r)

    return out


# speedup vs baseline: 3.3621x; 1.1392x over previous
import jax
import jax.numpy as jnp
from jax.experimental import pallas as pl
from jax.experimental.pallas import tpu as pltpu


def _bf(x):
    return x.astype(jnp.bfloat16)


def _fused_kernel(f_ref, d_ref, c_ref, w_ref, w2_ref, bcat_ref, b2_ref,
                  o_ref, fw_ref, y_ref, dbf_ref):
    p = pl.program_id(0)
    i = pl.program_id(1)
    bi = f_ref.shape[0]
    h = fw_ref.shape[1] // 2

    @pl.when(p == 0)
    def _phase_fw():
        fw_ref[pl.ds(i * bi, bi), :] = _bf(
            jnp.dot(_bf(f_ref[...]), _bf(w_ref[...]),
                    preferred_element_type=jnp.float32))

    @pl.when(p == 1)
    def _phase_agg():
        d = d_ref[...]
        dbf = _bf(d)
        dbf_ref[pl.ds(i * bi, bi), :] = dbf
        cd = c_ref[...] * d
        x1 = jnp.dot(dbf, fw_ref[:, :h],
                     preferred_element_type=jnp.float32)
        x2 = jnp.dot(_bf(cd), fw_ref[:, h:],
                     preferred_element_type=jnp.float32)
        x = jnp.maximum(jnp.concatenate([x1, x2], axis=1) + bcat_ref[...],
                        0.0)
        y_ref[pl.ds(i * bi, bi), :] = _bf(
            jnp.dot(_bf(x), _bf(w2_ref[...]),
                    preferred_element_type=jnp.float32))

    @pl.when(p == 2)
    def _phase_out():
        o_ref[...] = jnp.dot(dbf_ref[pl.ds(i * bi, bi), :], y_ref[...],
                             preferred_element_type=jnp.float32) + b2_ref[...]


def kernel(features, dist, adj_relative_cos, W1, b1, Wa, ba, W2, b2):
    n, in_dim = features.shape
    hid = W1.shape[1]
    out_dim = W2.shape[1]
    two_h = hid + in_dim

    wcat = jnp.concatenate([W1, Wa], axis=1)
    bcat = jnp.concatenate([b1, ba]).reshape(1, -1)
    b2r = b2.reshape(1, -1)

    bi = 256
    steps = n // bi

    out = pl.pallas_call(
        _fused_kernel,
        grid=(3, steps),
        in_specs=[
            pl.BlockSpec((bi, in_dim),
                         lambda p, i: (jnp.where(p == 0, i, 0), 0)),
            pl.BlockSpec((bi, n),
                         lambda p, i: (jnp.where(p == 0, 0,
                                       jnp.where(p == 1, i, n // bi - 1)), 0)),
            pl.BlockSpec((bi, n),
                         lambda p, i: (jnp.where(p == 1, i,
                                       jnp.where(p == 0, 0, n // bi - 1)), 0)),
            pl.BlockSpec((in_dim, two_h), lambda p, i: (0, 0)),
            pl.BlockSpec((two_h, out_dim), lambda p, i: (0, 0)),
            pl.BlockSpec((1, two_h), lambda p, i: (0, 0)),
            pl.BlockSpec((1, out_dim), lambda p, i: (0, 0)),
        ],
        out_specs=pl.BlockSpec((bi, out_dim),
                               lambda p, i: (jnp.where(p == 2, i, 0), 0)),
        out_shape=jax.ShapeDtypeStruct((n, out_dim), jnp.float32),
        scratch_shapes=[
            pltpu.VMEM((n, two_h), jnp.bfloat16),
            pltpu.VMEM((n, out_dim), jnp.bfloat16),
            pltpu.VMEM((n, n), jnp.bfloat16),
        ],
        compiler_params=pltpu.CompilerParams(
            dimension_semantics=("arbitrary", "arbitrary")),
    )(features, dist, adj_relative_cos, wcat, W2, bcat, b2r)

    return out


# speedup vs baseline: 3.9436x; 1.1730x over previous
import jax
import jax.numpy as jnp
from jax.experimental import pallas as pl
from jax.experimental.pallas import tpu as pltpu


def _bf(x):
    return x.astype(jnp.bfloat16)


_FW_STEPS = 4
_AGG_STEPS = 16
_OUT_STEPS = 4


def _fused_kernel(f_ref, d_ref, c_ref, w_ref, w2_ref, bcat_ref, b2_ref,
                  o_ref, fw_ref, y_ref, dbf_ref):
    i = pl.program_id(0)
    h = fw_ref.shape[1] // 2
    bf_rows = f_ref.shape[0]
    bi = d_ref.shape[0]
    bo = o_ref.shape[0]

    @pl.when(i < _FW_STEPS)
    def _phase_fw():
        s = jnp.minimum(i, _FW_STEPS - 1)
        fw_ref[pl.ds(s * bf_rows, bf_rows), :] = _bf(
            jnp.dot(_bf(f_ref[...]), _bf(w_ref[...]),
                    preferred_element_type=jnp.float32))

    @pl.when((i >= _FW_STEPS) & (i < _FW_STEPS + _AGG_STEPS))
    def _phase_agg():
        s = jnp.clip(i - _FW_STEPS, 0, _AGG_STEPS - 1)
        d = d_ref[...]
        dbf = _bf(d)
        dbf_ref[pl.ds(s * bi, bi), :] = dbf
        cd = c_ref[...] * d
        x1 = jnp.dot(dbf, fw_ref[:, :h],
                     preferred_element_type=jnp.float32)
        x2 = jnp.dot(_bf(cd), fw_ref[:, h:],
                     preferred_element_type=jnp.float32)
        x = jnp.maximum(jnp.concatenate([x1, x2], axis=1) + bcat_ref[...],
                        0.0)
        y_ref[pl.ds(s * bi, bi), :] = _bf(
            jnp.dot(_bf(x), _bf(w2_ref[...]),
                    preferred_element_type=jnp.float32))

    @pl.when(i >= _FW_STEPS + _AGG_STEPS)
    def _phase_out():
        s = jnp.maximum(i - (_FW_STEPS + _AGG_STEPS), 0)
        o_ref[...] = jnp.dot(dbf_ref[pl.ds(s * bo, bo), :], y_ref[...],
                             preferred_element_type=jnp.float32) + b2_ref[...]


def kernel(features, dist, adj_relative_cos, W1, b1, Wa, ba, W2, b2):
    n, in_dim = features.shape
    hid = W1.shape[1]
    out_dim = W2.shape[1]
    two_h = hid + in_dim

    wcat = jnp.concatenate([W1, Wa], axis=1)
    bcat = jnp.concatenate([b1, ba]).reshape(1, -1)
    b2r = b2.reshape(1, -1)

    bf_rows = n // _FW_STEPS
    bi = n // _AGG_STEPS
    bo = n // _OUT_STEPS
    steps = _FW_STEPS + _AGG_STEPS + _OUT_STEPS

    out = pl.pallas_call(
        _fused_kernel,
        grid=(steps,),
        in_specs=[
            pl.BlockSpec((bf_rows, in_dim),
                         lambda i: (jnp.minimum(i, _FW_STEPS - 1), 0)),
            pl.BlockSpec((bi, n),
                         lambda i: (jnp.clip(i - _FW_STEPS, 0,
                                             _AGG_STEPS - 1), 0)),
            pl.BlockSpec((bi, n),
                         lambda i: (jnp.clip(i - _FW_STEPS, 0,
                                             _AGG_STEPS - 1), 0)),
            pl.BlockSpec((in_dim, two_h), lambda i: (0, 0)),
            pl.BlockSpec((two_h, out_dim), lambda i: (0, 0)),
            pl.BlockSpec((1, two_h), lambda i: (0, 0)),
            pl.BlockSpec((1, out_dim), lambda i: (0, 0)),
        ],
        out_specs=pl.BlockSpec(
            (bo, out_dim),
            lambda i: (jnp.maximum(i - (_FW_STEPS + _AGG_STEPS), 0), 0)),
        out_shape=jax.ShapeDtypeStruct((n, out_dim), jnp.float32),
        scratch_shapes=[
            pltpu.VMEM((n, two_h), jnp.bfloat16),
            pltpu.VMEM((n, out_dim), jnp.bfloat16),
            pltpu.VMEM((n, n), jnp.bfloat16),
        ],
        compiler_params=pltpu.CompilerParams(
            dimension_semantics=("arbitrary",)),
    )(features, dist, adj_relative_cos, wcat, W2, bcat, b2r)

    return out
